# Initial kernel scaffold; baseline (speedup 1.0000x reference)
import jax
import jax.numpy as jnp
from jax.experimental import pallas as pl


def kernel(x, edge_index, batch, edge_weights, W1, b1, W2, b2, W3, b3, W4, b4):
    raise NotImplementedError("write your pallas kernel here")


# speedup vs baseline: 3.5192x; 3.5192x over previous
import functools

import jax
import jax.numpy as jnp
from jax import lax
from jax.experimental import pallas as pl
from jax.experimental.pallas import tpu as pltpu
from jax.experimental.pallas import tpu_sc as plsc

N = 10000
NPAD = 10240
D = 128
G = 64
NC = 2
NS = 16
CHUNK = 128
ROW_BLK = 1000
N_BLKS = N // ROW_BLK


def _make_edge_agg(t_edges):
    n_chunks = t_edges // CHUNK
    rows_per_tile = NPAD // NS
    zero_rows = 128

    mesh = plsc.VectorSubcoreMesh(core_axis_name="c", subcore_axis_name="s")

    @functools.partial(
        pl.kernel,
        mesh=mesh,
        out_type=jax.ShapeDtypeStruct((NC * NPAD, D), jnp.float32),
        scratch_types=[
            pltpu.VMEM((CHUNK,), jnp.int32),
            pltpu.VMEM((CHUNK,), jnp.int32),
            pltpu.VMEM((CHUNK,), jnp.float32),
            pltpu.VMEM((CHUNK, D), jnp.float32),
            pltpu.VMEM_SHARED((NPAD, D), jnp.float32),
            pltpu.SemaphoreType.DMA,
        ],
    )
    def edge_agg(hw_hbm, src_hbm, dst_hbm, w_hbm, out_hbm,
                 src_v, dst_v, w_v, rows_v, acc, sem):
        cid = lax.axis_index("c")
        sid = lax.axis_index("s")
        wid = cid * NS + sid

        def zrow(r, _):
            for k in range(D // 16):
                rows_v[r, pl.ds(k * 16, 16)] = jnp.zeros((16,), jnp.float32)
            return 0
        lax.fori_loop(0, CHUNK, zrow, 0)
        for j in range(rows_per_tile // zero_rows):
            pltpu.sync_copy(
                rows_v.at[pl.ds(0, zero_rows)],
                acc.at[pl.ds(sid * rows_per_tile + j * zero_rows, zero_rows)],
            )
        plsc.subcore_barrier()

        def chunk_body(ci, _):
            base = pl.multiple_of(wid * t_edges + ci * CHUNK, CHUNK)
            pltpu.sync_copy(src_hbm.at[pl.ds(base, CHUNK)], src_v)
            pltpu.sync_copy(dst_hbm.at[pl.ds(base, CHUNK)], dst_v)
            pltpu.sync_copy(w_hbm.at[pl.ds(base, CHUNK)], w_v)
            pltpu.async_copy(hw_hbm.at[src_v], rows_v, sem).wait()

            def grp_scale(g, _):
                w16 = w_v[pl.ds(g * 16, 16)]
                for j in range(16):
                    ws = w16[j]
                    e = g * 16 + j
                    for k in range(D // 16):
                        rows_v[e, pl.ds(k * 16, 16)] = (
                            rows_v[e, pl.ds(k * 16, 16)] * ws)
                return 0
            lax.fori_loop(0, CHUNK // 16, grp_scale, 0)

            pltpu.sync_copy(rows_v, acc.at[dst_v], add=True)
            return 0
        lax.fori_loop(0, n_chunks, chunk_body, 0)

        plsc.subcore_barrier()

        r0 = sid * rows_per_tile
        pltpu.sync_copy(
            acc.at[pl.ds(r0, rows_per_tile)],
            out_hbm.at[pl.ds(cid * NPAD + r0, rows_per_tile)],
        )

    return edge_agg


def _mm_kernel(x_ref, w_ref, o_ref):
    o_ref[...] = jnp.dot(x_ref[...], w_ref[...],
                         preferred_element_type=jnp.float32)


def _tc_matmul(x, w):
    return pl.pallas_call(
        _mm_kernel,
        grid=(N_BLKS,),
        in_specs=[
            pl.BlockSpec((ROW_BLK, D), lambda i: (i, 0)),
            pl.BlockSpec((D, D), lambda i: (0, 0)),
        ],
        out_specs=pl.BlockSpec((ROW_BLK, D), lambda i: (i, 0)),
        out_shape=jax.ShapeDtypeStruct((N, D), jnp.float32),
    )(x, w)


def _fused_kernel(z0_ref, z1_ref, hw_ref, b_ref, w_ref, o_ref):
    h = jax.nn.relu(z0_ref[...] + z1_ref[...] + hw_ref[...] + b_ref[...])
    o_ref[...] = jnp.dot(h, w_ref[...], preferred_element_type=jnp.float32)


def _tc_fused_layer(z0, z1, hw, b, w):
    return pl.pallas_call(
        _fused_kernel,
        grid=(N_BLKS,),
        in_specs=[
            pl.BlockSpec((ROW_BLK, D), lambda i: (i, 0)),
            pl.BlockSpec((ROW_BLK, D), lambda i: (i, 0)),
            pl.BlockSpec((ROW_BLK, D), lambda i: (i, 0)),
            pl.BlockSpec((1, D), lambda i: (0, 0)),
            pl.BlockSpec((D, D), lambda i: (0, 0)),
        ],
        out_specs=pl.BlockSpec((ROW_BLK, D), lambda i: (i, 0)),
        out_shape=jax.ShapeDtypeStruct((N, D), jnp.float32),
    )(z0, z1, hw, b, w)


def _pool_kernel(z0_ref, z1_ref, hw_ref, b_ref, batch_ref, w4_ref, b4_ref,
                 o_ref, sums_ref, cnts_ref):
    i = pl.program_id(0)

    @pl.when(i == 0)
    def _():
        sums_ref[...] = jnp.zeros_like(sums_ref)
        cnts_ref[...] = jnp.zeros_like(cnts_ref)

    h = jax.nn.relu(z0_ref[...] + z1_ref[...] + hw_ref[...] + b_ref[...])
    bids = batch_ref[0]
    gids = lax.broadcasted_iota(jnp.int32, (G, ROW_BLK), 0)
    mask = (bids == gids).astype(jnp.float32)
    sums_ref[...] += jnp.dot(mask, h, preferred_element_type=jnp.float32)
    cnts_ref[...] += jnp.sum(mask, axis=1, keepdims=True)

    @pl.when(i == N_BLKS - 1)
    def _():
        pooled = sums_ref[...] / jnp.maximum(cnts_ref[...], 1.0)
        o_ref[...] = jnp.dot(pooled, w4_ref[...],
                             preferred_element_type=jnp.float32) + b4_ref[...]


def _tc_pool(z0, z1, hw, b, batch3d, w4, b4):
    return pl.pallas_call(
        _pool_kernel,
        grid=(N_BLKS,),
        in_specs=[
            pl.BlockSpec((ROW_BLK, D), lambda i: (i, 0)),
            pl.BlockSpec((ROW_BLK, D), lambda i: (i, 0)),
            pl.BlockSpec((ROW_BLK, D), lambda i: (i, 0)),
            pl.BlockSpec((1, D), lambda i: (0, 0)),
            pl.BlockSpec((1, 1, ROW_BLK), lambda i: (i, 0, 0)),
            pl.BlockSpec((D, D), lambda i: (0, 0)),
            pl.BlockSpec((1, D), lambda i: (0, 0)),
        ],
        out_specs=pl.BlockSpec((G, D), lambda i: (0, 0)),
        out_shape=jax.ShapeDtypeStruct((G, D), jnp.float32),
        scratch_shapes=[
            pltpu.VMEM((G, D), jnp.float32),
            pltpu.VMEM((G, D), jnp.float32),
        ],
    )(z0, z1, hw, b, batch3d, w4, b4)


def kernel(x, edge_index, batch, edge_weights, W1, b1, W2, b2, W3, b3, W4, b4):
    E = edge_index.shape[1]
    n_workers = NC * NS
    t_edges = -(-E // (n_workers * CHUNK)) * CHUNK
    e_pad = n_workers * t_edges

    src = edge_index[0].astype(jnp.int32)
    dst = edge_index[1].astype(jnp.int32)
    w = edge_weights.astype(jnp.float32)
    pad = e_pad - E
    if pad:
        src = jnp.concatenate([src, jnp.zeros((pad,), jnp.int32)])
        dst = jnp.concatenate([dst, jnp.zeros((pad,), jnp.int32)])
        w = jnp.concatenate([w, jnp.zeros((pad,), jnp.float32)])

    edge_agg = _make_edge_agg(t_edges)

    b1r = b1.reshape(1, D)
    b2r = b2.reshape(1, D)
    b3r = b3.reshape(1, D)
    b4r = b4.reshape(1, D)
    batch3d = batch.astype(jnp.int32).reshape(N_BLKS, 1, ROW_BLK)

    hw1 = _tc_matmul(x, W1)
    z1 = edge_agg(hw1, src, dst, w)
    hw2 = _tc_fused_layer(z1[:N], z1[NPAD:NPAD + N], hw1, b1r, W2)
    z2 = edge_agg(hw2, src, dst, w)
    hw3 = _tc_fused_layer(z2[:N], z2[NPAD:NPAD + N], hw2, b2r, W3)
    z3 = edge_agg(hw3, src, dst, w)
    return _tc_pool(z3[:N], z3[NPAD:NPAD + N], hw3, b3r, batch3d, W4, b4r)
